# baseline (speedup 1.0000x reference)
import jax
import jax.numpy as jnp
import numpy as np

import sc_part

_N, _D = 8, 2048


@jax.jit
def kernel(x, W, b):
    xb = x.astype(jnp.bfloat16).astype(jnp.float32)
    sc = sc_part.sc_logits(xb, W.reshape(_N * _D, _D))

    logits = jnp.einsum('nd,nkd->nk', x, W) + b
    logits = logits.at[:, :sc_part.KSC].set(sc + b[:, :sc_part.KSC])

    u = jax.random.uniform(jax.random.key(42), logits.shape, dtype=jnp.float32)
    gumbel = -jnp.log(-jnp.log(u + 1e-20) + 1e-20)
    actions = jnp.argmax(logits + gumbel, axis=-1)
    logp = jax.nn.log_softmax(logits, axis=-1)
    action_log_probs = jnp.take_along_axis(logp, actions[:, None], axis=-1)
    return actions[:, None].astype(jnp.int64), action_log_probs


# speedup vs baseline: 1.7416x; 1.7416x over previous
import jax
import jax.numpy as jnp
import numpy as np
from jax.experimental import pallas as pl
from jax.experimental.pallas import tpu as pltpu

_N, _D = 8, 2048
_R = 256
_NBUF = 4
_CPA = _D // _R
_C = _N * _CPA
_NEG = -1e30
_BIG = 2 ** 30


def _np_threefry2x32(k1, k2, x1, x2):
    def rotl(v, r):
        return ((v << np.uint32(r)) | (v >> np.uint32(32 - r))).astype(np.uint32)

    rots = ([13, 15, 26, 6], [17, 29, 16, 24])
    ks = [np.uint32(k1), np.uint32(k2),
          np.uint32(k1) ^ np.uint32(k2) ^ np.uint32(0x1BD11BDA)]
    x = [x1.astype(np.uint32) + ks[0], x2.astype(np.uint32) + ks[1]]

    def rounds(x, rs):
        for r in rs:
            x[0] = (x[0] + x[1]).astype(np.uint32)
            x[1] = x[0] ^ rotl(x[1], r)
        return x

    old = np.seterr(over="ignore")
    for i, (ka, kb) in enumerate([(1, 2), (2, 0), (0, 1), (1, 2), (2, 0)]):
        x = rounds(x, rots[i % 2])
        x = [x[0] + ks[ka], x[1] + ks[kb] + np.uint32(i + 1)]
    np.seterr(**old)
    return x[0], x[1]


def _make_gumbel() -> np.ndarray:
    n = _N * _D
    idx = np.arange(n, dtype=np.uint64)
    c1 = (idx >> np.uint64(32)).astype(np.uint32)
    c2 = (idx & np.uint64(0xFFFFFFFF)).astype(np.uint32)
    b1, b2 = _np_threefry2x32(np.uint32(0), np.uint32(42), c1, c2)
    bits = (b1 ^ b2).astype(np.uint32)
    fb = (bits >> np.uint32(9)) | np.uint32(0x3F800000)
    u = (fb.view(np.float32) - np.float32(1.0)).reshape(_N, _D)
    eps = np.float32(1e-20)
    return -np.log(-np.log(u + eps) + eps)


_GUMBEL = _make_gumbel()


def _fused_body(x_ref, w_hbm, b_ref, g_ref, act_ref, logp_ref,
                bufs, acc, sems):
    def make(c, slot):
        a = c // _CPA
        r = c % _CPA
        return pltpu.make_async_copy(
            w_hbm.at[a, pl.ds(r * _R, _R), :], bufs.at[slot], sems.at[slot])

    for s in range(_NBUF):
        make(s, s).start()

    def loop_body(c, carry):
        slot = jax.lax.rem(c, _NBUF)
        a = c // _CPA
        r = c % _CPA
        make(c, slot).wait()

        chunk = jax.lax.dot_general(
            x_ref[pl.ds(a, 1), :], bufs[slot],
            dimension_numbers=(((1,), (1,)), ((), ())),
            preferred_element_type=jnp.float32,
            precision=jax.lax.Precision.DEFAULT,
        )
        acc[pl.ds(r, 1), :] = chunk

        @pl.when(c + _NBUF < _C)
        def _():
            make(c + _NBUF, slot).start()

        @pl.when(r == _CPA - 1)
        def _finalize():
            logits = acc[...] + b_ref[a]
            pert = logits + g_ref[a]
            kvec = (_R * jax.lax.broadcasted_iota(jnp.int32, (_CPA, _R), 0)
                    + jax.lax.broadcasted_iota(jnp.int32, (_CPA, _R), 1))
            m = jnp.max(pert, axis=(0, 1), keepdims=True)
            winidx = jnp.min(jnp.where(pert == m, kvec, _BIG),
                             axis=(0, 1), keepdims=True)
            blog = jnp.max(jnp.where(kvec == winidx, logits, _NEG),
                           axis=(0, 1), keepdims=True)
            lse = jnp.log(jnp.sum(jnp.exp(logits), axis=(0, 1), keepdims=True))
            act_ref[pl.ds(a, 1), :] = jnp.broadcast_to(winidx[0], (1, 128))
            logp_ref[pl.ds(a, 1), :] = jnp.broadcast_to((blog - lse)[0], (1, 128))

        return carry

    jax.lax.fori_loop(0, _C, loop_body, 0)


@jax.jit
def kernel(x, W, b):
    g3 = jnp.asarray(_GUMBEL).reshape(_N, _CPA, _R)

    acts, logps = pl.pallas_call(
        _fused_body,
        in_specs=[
            pl.BlockSpec(memory_space=pltpu.MemorySpace.VMEM),
            pl.BlockSpec(memory_space=pl.ANY),
            pl.BlockSpec(memory_space=pltpu.MemorySpace.VMEM),
            pl.BlockSpec(memory_space=pltpu.MemorySpace.VMEM),
        ],
        out_specs=[
            pl.BlockSpec(memory_space=pltpu.MemorySpace.VMEM),
            pl.BlockSpec(memory_space=pltpu.MemorySpace.VMEM),
        ],
        out_shape=[
            jax.ShapeDtypeStruct((_N, 128), jnp.int32),
            jax.ShapeDtypeStruct((_N, 128), jnp.float32),
        ],
        scratch_shapes=[
            pltpu.VMEM((_NBUF, _R, _D), jnp.float32),
            pltpu.VMEM((_CPA, _R), jnp.float32),
            pltpu.SemaphoreType.DMA((_NBUF,)),
        ],
    )(x, W, b.reshape(_N, _CPA, _R), g3)

    actions = acts[:, :1].astype(jnp.int64)
    return actions, logps[:, :1]
